# baseline (speedup 1.0000x reference)
import jax
import jax.numpy as jnp
from jax.experimental import pallas as pl
from jax.experimental.pallas import tpu as pltpu

_EPS = 1e-5


def _scatter_body(idx_ref, feats_ref, out_ref):
    @pl.when(pl.program_id(0) == 0)
    def _init():
        out_ref[...] = jnp.zeros_like(out_ref)

    def body(i, carry):
        r = idx_ref[0, 0, i]
        out_ref[pl.ds(r, 1), :] += feats_ref[pl.ds(i, 1), :]
        return carry

    jax.lax.fori_loop(0, idx_ref.shape[2], body, 0)


def _scatter(feats, coords, dims, chunk=512):
    d, h, w = dims
    hp, wp = h + 2, w + 2
    n_rows = d * hp * wp
    n, c = feats.shape
    caug = c + 8
    aug = jnp.concatenate(
        [feats, jnp.ones((n, 1), jnp.float32), jnp.zeros((n, 7), jnp.float32)],
        axis=1)
    idx = (coords[:, 1] * (hp * wp) + (coords[:, 2] + 1) * wp
           + (coords[:, 3] + 1)).astype(jnp.int32)
    steps = -(-n // chunk)
    pad = steps * chunk - n
    if pad:
        aug = jnp.pad(aug, ((0, pad), (0, 0)))
        idx = jnp.pad(idx, (0, pad))
    idx = idx.reshape(steps, 1, chunk)
    return pl.pallas_call(
        _scatter_body,
        grid=(steps,),
        in_specs=[
            pl.BlockSpec((1, 1, chunk), lambda i: (i, 0, 0),
                         memory_space=pltpu.SMEM),
            pl.BlockSpec((chunk, caug), lambda i: (i, 0)),
        ],
        out_specs=pl.BlockSpec((n_rows, caug), lambda i: (0, 0)),
        out_shape=jax.ShapeDtypeStruct((n_rows, caug), jnp.float32),
    )(idx, aug)


def _bn_vals(y, mask, g, b):
    n = jnp.maximum(jnp.sum(mask), 1.0)
    mean = jnp.sum(y * mask, axis=0, keepdims=True) / n
    xc = (y - mean) * mask
    var = jnp.sum(xc * xc, axis=0, keepdims=True) / n
    return (y - mean) * jax.lax.rsqrt(var + _EPS) * g + b


def _interior_col(dims):
    d, h, w = dims
    hp, wp = h + 2, w + 2
    n_rows = d * hp * wp
    p = jax.lax.broadcasted_iota(jnp.int32, (n_rows, 1), 0)
    rem = p % (hp * wp)
    hh = rem // wp
    ww = rem % wp
    ok = (hh >= 1) & (hh <= h) & (ww >= 1) & (ww <= w)
    return ok.astype(jnp.float32)


def _lateral(grid_aug, w, g, b, c_in, n_rows):
    def body(xg_ref, w_ref, g_ref, b_ref, out_ref, m_ref):
        x = xg_ref[:, :c_in]
        cnt = xg_ref[:, c_in:c_in + 1]
        mask = (cnt > 0.0).astype(jnp.float32)
        y = jnp.dot(x, w_ref[...], preferred_element_type=jnp.float32)
        ybn = _bn_vals(y, mask, g_ref[...], b_ref[...])
        out_ref[...] = ybn * mask
        m_ref[...] = mask

    return pl.pallas_call(
        body,
        out_shape=[
            jax.ShapeDtypeStruct((n_rows, 128), jnp.float32),
            jax.ShapeDtypeStruct((n_rows, 1), jnp.float32),
        ],
    )(grid_aug, w, g.reshape(1, 128), b.reshape(1, 128))


def _taps(dims):
    d, h, w = dims
    hp, wp = h + 2, w + 2
    n_rows = d * hp * wp
    out = []
    for kd in range(3):
        for kh in range(3):
            for kw in range(3):
                o = (kd - 1) * hp * wp + (kh - 1) * wp + (kw - 1)
                a = max(0, -o)
                m = n_rows - a - max(0, o)
                out.append((kd * 9 + kh * 3 + kw, o, a, m))
    return n_rows, out


def _conv27(x, w, dims):
    n_rows, taps = _taps(dims)

    def body(x_ref, w_ref, acc_ref):
        acc_ref[...] = jnp.zeros_like(acc_ref)
        xv = x_ref[...]
        for t, o, a, m in taps:
            z = jnp.dot(xv, w_ref[t], preferred_element_type=jnp.float32)
            acc_ref[pl.ds(a, m), :] += jax.lax.slice(
                z, (a + o, 0), (a + o + m, 128))

    return pl.pallas_call(
        body,
        out_shape=jax.ShapeDtypeStruct((n_rows, 128), jnp.float32),
    )(x, w.reshape(27, 128, 128))


def _maskconv27(md, dims):
    n_rows, taps = _taps(dims)

    def body(md_ref, mc_ref):
        mc_ref[...] = jnp.zeros_like(mc_ref)
        mv = md_ref[...]
        for _, o, a, m in taps:
            mc_ref[pl.ds(a, m), :] += jax.lax.slice(
                mv, (a + o, 0), (a + o + m, 1))

    return pl.pallas_call(
        body,
        out_shape=jax.ShapeDtypeStruct((n_rows, 1), jnp.float32),
    )(md)


def _upsample(xd, md, skip, skipm, w, g, b, dims):
    d, h, ww_ = dims
    n_rows = d * (h + 2) * (ww_ + 2)
    acc = _conv27(xd, w, dims)
    mc = _maskconv27(md, dims)

    def merge_body(mc_ref, skipm_ref, outm_ref):
        mask = ((mc_ref[...] * _interior_col(dims)) > 0.0).astype(jnp.float32)
        outm_ref[...] = jnp.maximum(skipm_ref[...], mask)

    outm = pl.pallas_call(
        merge_body,
        out_shape=jax.ShapeDtypeStruct((n_rows, 1), jnp.float32),
    )(mc, skipm)

    def bnadd_body(acc_ref, mc_ref, skip_ref, g_ref, b_ref, out_ref):
        mask = ((mc_ref[...] * _interior_col(dims)) > 0.0).astype(jnp.float32)
        ybn = _bn_vals(acc_ref[...], mask, g_ref[...], b_ref[...])
        out_ref[...] = skip_ref[...] + jnp.maximum(ybn, 0.0) * mask

    out = pl.pallas_call(
        bnadd_body,
        out_shape=jax.ShapeDtypeStruct((n_rows, 128), jnp.float32),
    )(acc, mc, skip, g.reshape(1, 128), b.reshape(1, 128))
    return out, outm


def _final(x, mask, w, g, b, dims):
    d, h, ww_ = dims
    n_rows = d * (h + 2) * (ww_ + 2)
    acc = _conv27(x, w, dims)

    def body(acc_ref, m_ref, g_ref, b_ref, out_ref):
        m = m_ref[...]
        y = acc_ref[...] * m
        ybn = _bn_vals(y, m, g_ref[...], b_ref[...])
        out_ref[...] = jnp.maximum(ybn, 0.0) * m

    return pl.pallas_call(
        body,
        out_shape=jax.ShapeDtypeStruct((n_rows, 128), jnp.float32),
    )(acc, mask, g.reshape(1, 128), b.reshape(1, 128))


def _dilate(flat, d, h_in, w_in, c):
    h_out, w_out = 2 * h_in - 1, 2 * w_in - 1
    x = flat.reshape(d, h_in + 2, w_in + 2, c)[:, 1:1 + h_in, 1:1 + w_in, :]
    z = jnp.zeros((d, h_out, w_out, c), jnp.float32)
    z = z.at[:, ::2, ::2, :].set(x)
    z = jnp.pad(z, ((0, 0), (1, 1), (1, 1), (0, 0)))
    return z.reshape(d * (h_out + 2) * (w_out + 2), c)


def kernel(c2_feats, c3_feats, c4_feats,
           lat2_w, lat2_g, lat2_b, lat3_w, lat3_g, lat3_b,
           lat4_w, lat4_g, lat4_b, up43_w, up43_g, up43_b,
           up32_w, up32_g, up32_b, out_w, out_g, out_b,
           c2_coords, c3_coords, c4_coords):
    lv2, lv3, lv4 = (2, 93, 93), (2, 47, 47), (2, 24, 24)
    n2 = 2 * 95 * 95
    n3 = 2 * 49 * 49
    n4 = 2 * 26 * 26

    g2 = _scatter(c2_feats, c2_coords, lv2)
    g3 = _scatter(c3_feats, c3_coords, lv3)
    g4 = _scatter(c4_feats, c4_coords, lv4)

    p2, m2 = _lateral(g2, lat2_w, lat2_g, lat2_b, 64, n2)
    p3, m3 = _lateral(g3, lat3_w, lat3_g, lat3_b, 128, n3)
    p4, m4 = _lateral(g4, lat4_w, lat4_g, lat4_b, 256, n4)

    p3, m3 = _upsample(_dilate(p4, 2, 24, 24, 128), _dilate(m4, 2, 24, 24, 1),
                       p3, m3, up43_w, up43_g, up43_b, lv3)
    p2, m2 = _upsample(_dilate(p3, 2, 47, 47, 128), _dilate(m3, 2, 47, 47, 1),
                       p2, m2, up32_w, up32_g, up32_b, lv2)

    y = _final(p2, m2, out_w, out_g, out_b, lv2)
    out = y.reshape(2, 95, 95, 128)[:, 1:94, 1:94, :]
    return out[None]
